# baseline (speedup 1.0000x reference)
import functools

import jax
import jax.numpy as jnp
from jax.experimental import pallas as pl
from jax.experimental.pallas import tpu as pltpu

_CAPACITY = 0.5


def _score_body(hs_ref, w_ref, b_ref, out_ref):
    x = hs_ref[...].astype(jnp.bfloat16)
    wrep = w_ref[...]
    acc = jax.lax.dot_general(
        wrep, x, (((1,), (1,)), ((), ())),
        preferred_element_type=jnp.float32)
    logits = acc[0, :] + b_ref[0]
    out_ref[0, 0, :] = jax.nn.sigmoid(logits)


def _mask_body(k, w_ref, mask_ref):
    w = w_ref[...]
    keys = jax.lax.bitcast_convert_type(w, jnp.int32)
    B = w.shape[0]

    def body(i, t):
        bit = jax.lax.shift_left(jnp.int32(1), jnp.int32(30) - i)
        cand = t | bit
        cnt = jnp.sum((keys >= cand).astype(jnp.int32), axis=1, keepdims=True)
        return jnp.where(cnt >= k, cand, t)

    t = jax.lax.fori_loop(0, 31, body, jnp.zeros((B, 1), jnp.int32))
    thr = jax.lax.bitcast_convert_type(t, jnp.float32)
    mask_ref[...] = (w >= thr).astype(jnp.int8)


def kernel(hidden_states, W, b):
    B, S, D = hidden_states.shape
    k = max(1, int(_CAPACITY * S))

    BS = 1024
    n_blk = (B * S) // BS
    hs2 = hidden_states.reshape(B * S, D)
    wrep = jnp.broadcast_to(W.astype(jnp.bfloat16), (8, D))

    weights3 = pl.pallas_call(
        _score_body,
        grid=(n_blk,),
        in_specs=[
            pl.BlockSpec((BS, D), lambda i: (i, 0)),
            pl.BlockSpec((8, D), lambda i: (0, 0)),
            pl.BlockSpec((1,), lambda i: (0,)),
        ],
        out_specs=pl.BlockSpec((1, 1, BS), lambda i: (i, 0, 0)),
        out_shape=jax.ShapeDtypeStruct((n_blk, 1, BS), jnp.float32),
        compiler_params=pltpu.CompilerParams(
            dimension_semantics=("parallel",)),
    )(hs2, wrep, b)
    weights = weights3.reshape(B, S)

    mask_i8 = pl.pallas_call(
        functools.partial(_mask_body, k),
        out_shape=jax.ShapeDtypeStruct((B, S), jnp.int8),
    )(weights)

    return weights, mask_i8.astype(bool)


# speedup vs baseline: 1.0870x; 1.0870x over previous
import functools

import jax
import jax.numpy as jnp
from jax.experimental import pallas as pl
from jax.experimental.pallas import tpu as pltpu

_CAPACITY = 0.5


def _score_body(hs_a_ref, hs_b_ref, w_ref, b_ref, out_a_ref, out_b_ref):
    wrep = w_ref[...]
    bias = b_ref[0]
    for hs_ref, out_ref in ((hs_a_ref, out_a_ref), (hs_b_ref, out_b_ref)):
        x = hs_ref[0]
        acc = jax.lax.dot_general(
            wrep, x.astype(jnp.bfloat16), (((1,), (1,)), ((), ())),
            preferred_element_type=jnp.float32)
        out_ref[0, 0, :] = jax.nn.sigmoid(acc[0, :] + bias)


def _mask_body(k, w_ref, mask_ref):
    w = w_ref[...]
    keys = jax.lax.bitcast_convert_type(w, jnp.int32)
    B = w.shape[0]

    def body(i, t):
        bit = jax.lax.shift_left(jnp.int32(1), jnp.int32(30) - i)
        cand = t | bit
        cnt = jnp.sum((keys >= cand).astype(jnp.int32), axis=1, keepdims=True)
        return jnp.where(cnt >= k, cand, t)

    t = jax.lax.fori_loop(0, 31, body, jnp.zeros((B, 1), jnp.int32))
    thr = jax.lax.bitcast_convert_type(t, jnp.float32)
    mask_ref[...] = (w >= thr).astype(jnp.int8)


def kernel(hidden_states, W, b):
    B, S, D = hidden_states.shape
    k = max(1, int(_CAPACITY * S))

    BS = 2048
    n_blk = (B * S) // BS
    half = n_blk // 2
    hs3 = hidden_states.reshape(n_blk, BS, D)
    wrep = jnp.broadcast_to(W.astype(jnp.bfloat16), (8, D))

    out_a, out_b = pl.pallas_call(
        _score_body,
        grid=(half,),
        in_specs=[
            pl.BlockSpec((1, BS, D), lambda i: (i, 0, 0)),
            pl.BlockSpec((1, BS, D), lambda i: (i + half, 0, 0)),
            pl.BlockSpec((8, D), lambda i: (0, 0)),
            pl.BlockSpec((1,), lambda i: (0,)),
        ],
        out_specs=[
            pl.BlockSpec((1, 1, BS), lambda i: (i, 0, 0)),
            pl.BlockSpec((1, 1, BS), lambda i: (i, 0, 0)),
        ],
        out_shape=[
            jax.ShapeDtypeStruct((half, 1, BS), jnp.float32),
            jax.ShapeDtypeStruct((half, 1, BS), jnp.float32),
        ],
        compiler_params=pltpu.CompilerParams(
            dimension_semantics=("parallel",)),
    )(hs3, hs3, wrep, b)
    weights = jnp.concatenate(
        [out_a.reshape(-1), out_b.reshape(-1)]).reshape(B, S)

    mask_i8 = pl.pallas_call(
        functools.partial(_mask_body, k),
        out_shape=jax.ShapeDtypeStruct((B, S), jnp.int8),
    )(weights)

    return weights, mask_i8.astype(bool)


# speedup vs baseline: 1.1777x; 1.0834x over previous
import functools

import jax
import jax.numpy as jnp
from jax.experimental import pallas as pl
from jax.experimental.pallas import tpu as pltpu

_CAPACITY = 0.5


def _fused_body(k, n_blk, blk_per_row, hs_ref, w_ref, b_ref,
                out_ref, mask_ref, wacc_ref):
    i = pl.program_id(0)
    BS = out_ref.shape[2]

    wrep = w_ref[...]
    x = hs_ref[0].astype(jnp.bfloat16)
    acc = jax.lax.dot_general(
        wrep, x, (((1,), (1,)), ((), ())),
        preferred_element_type=jnp.float32)
    wts = jax.nn.sigmoid(acc[0:1, :] + b_ref[0])
    out_ref[0, 0, :] = wts[0]

    row = i // blk_per_row
    off = (i % blk_per_row) * BS
    wacc_ref[pl.ds(row, 1), pl.ds(off, BS)] = wts

    @pl.when(i == n_blk - 1)
    def _select():
        w = wacc_ref[...]
        keys = jax.lax.bitcast_convert_type(w, jnp.int32)
        nrow = w.shape[0]

        def body(j, t):
            bit = jax.lax.shift_left(jnp.int32(1), jnp.int32(30) - j)
            cand = t | bit
            cnt = jnp.sum((keys >= cand).astype(jnp.int32),
                          axis=1, keepdims=True)
            return jnp.where(cnt >= k, cand, t)

        t = jax.lax.fori_loop(0, 31, body, jnp.zeros((nrow, 1), jnp.int32))
        thr = jax.lax.bitcast_convert_type(t, jnp.float32)
        mask_ref[...] = (w >= thr).astype(jnp.int8)


def kernel(hidden_states, W, b):
    B, S, D = hidden_states.shape
    k = max(1, int(_CAPACITY * S))

    BS = 2048
    n_blk = (B * S) // BS
    blk_per_row = S // BS
    hs3 = hidden_states.reshape(n_blk, BS, D)
    wrep = jnp.broadcast_to(W.astype(jnp.bfloat16), (8, D))

    weights3, mask_i8 = pl.pallas_call(
        functools.partial(_fused_body, k, n_blk, blk_per_row),
        grid=(n_blk,),
        in_specs=[
            pl.BlockSpec((1, BS, D), lambda i: (i, 0, 0)),
            pl.BlockSpec((8, D), lambda i: (0, 0)),
            pl.BlockSpec((1,), lambda i: (0,)),
        ],
        out_specs=[
            pl.BlockSpec((1, 1, BS), lambda i: (i, 0, 0)),
            pl.BlockSpec((B, S), lambda i: (0, 0)),
        ],
        out_shape=[
            jax.ShapeDtypeStruct((n_blk, 1, BS), jnp.float32),
            jax.ShapeDtypeStruct((B, S), jnp.int8),
        ],
        scratch_shapes=[pltpu.VMEM((B, S), jnp.float32)],
        compiler_params=pltpu.CompilerParams(
            dimension_semantics=("arbitrary",)),
    )(hs3, wrep, b)

    return weights3.reshape(B, S), mask_i8.astype(bool)
